# baseline (speedup 1.0000x reference)
import jax
import jax.numpy as jnp
from jax import lax
from jax.experimental import pallas as pl
from jax.experimental.pallas import tpu as pltpu
from jax.experimental.pallas import tpu_sc as plsc

N = 1000
NPAD = 1024
B = 16384
NC = 2
NS = 16
NW = NC * NS
BPW = B // NW
LANES = 16
CHUNK = 128
NCHUNK = BPW // CHUNK
IROWS = 2 * BPW // CHUNK


H0 = 512
H1 = N - H0


def _lse_fold_body(wc_hbm, wa_ref, in_hbm, t2_hbm, ab_hbm, wc_v, t2a_v, t2b_v,
                   sem_in, sem_out):
    del in_hbm, ab_hbm
    cp_a = pltpu.async_copy(wc_hbm.at[pl.ds(0, H0)], wc_v.at[pl.ds(0, H0)],
                            sem_in)
    cp_b = pltpu.async_copy(wc_hbm.at[pl.ds(H0, H1)], wc_v.at[pl.ds(H0, H1)],
                            sem_in)

    wa_all = wa_ref[...]
    ma = jnp.max(wa_all)
    sa = jnp.sum(jnp.exp(wa_all - ma))
    lse_a = ma + jnp.log(sa)

    def fold_half(row0, nrows, stage_ref):
        x = wc_v[pl.ds(row0, nrows), :]
        m = jnp.max(x, axis=1, keepdims=True)
        s = jnp.sum(jnp.exp(x - m), axis=1, keepdims=True)
        lse_c = m + jnp.log(s)
        t2 = x + (wa_ref[pl.ds(row0, nrows), :] - lse_a - lse_c)
        t2p = jnp.concatenate(
            [t2, jnp.zeros((nrows, NPAD - N), jnp.float32)], axis=1)
        for rg in range(nrows // 8):
            for tj in range(NPAD // 128):
                stage_ref[pl.ds(rg * 64 + tj * 8, 8), :] = (
                    t2p[rg * 8:(rg + 1) * 8, tj * 128:(tj + 1) * 128])

    cp_a.wait()
    fold_half(0, H0, t2a_v)
    cp_oa = pltpu.async_copy(t2a_v, t2_hbm.at[pl.ds(0, H0 * 8)], sem_out)
    cp_b.wait()
    fold_half(H0, H1, t2b_v)
    cp_ob = pltpu.async_copy(t2b_v, t2_hbm.at[pl.ds(H0 * 8, H1 * 8)], sem_out)
    cp_oa.wait()
    cp_ob.wait()


def _lane_shuffle(v, idx):
    return lax.gather(
        v, idx[:, None],
        lax.GatherDimensionNumbers(
            offset_dims=(), collapsed_slice_dims=(0,), start_index_map=(0,)),
        (1,),
        mode=lax.GatherScatterMode.PROMISE_IN_BOUNDS)


def _sc_body(in_hbm, t2_hbm, out_hbm, iv_v, idx_v, out_v, sem, gsem):
    wid = lax.axis_index("s") * NC + lax.axis_index("c")

    pltpu.async_copy(in_hbm.at[pl.ds(wid * IROWS, IROWS)], iv_v, sem).wait()

    lane = lax.iota(jnp.int32, LANES)
    rot1 = lax.bitwise_and(lane + 1, LANES - 1)
    compact = lax.bitwise_and(lane * 2, LANES - 1)
    low_half = lane < (LANES // 2)

    for i in range(BPW // LANES):
        q1, t1 = (2 * i) // 8, (2 * i) % 8
        q2, t2 = (2 * i + 1) // 8, (2 * i + 1) % 8
        v1 = iv_v[q1, pl.ds(t1 * LANES, LANES)]
        v2 = iv_v[q2, pl.ds(t2 * LANES, LANES)]
        f1 = (v1 >> 3) * 8192 + (v1 & 7) * 128
        g1 = (v1 >> 7) * 1024 + (v1 & 127)
        f2 = (v2 >> 3) * 8192 + (v2 & 7) * 128
        g2 = (v2 >> 7) * 1024 + (v2 & 127)
        u1 = f1 + _lane_shuffle(g1, rot1)
        u2 = f2 + _lane_shuffle(g2, rot1)
        off = jnp.where(low_half,
                        _lane_shuffle(u1, compact),
                        _lane_shuffle(u2, compact))
        idx_v[i // 8, pl.ds((i % 8) * LANES, LANES)] = off

    gathers = [
        pltpu.async_copy(t2_hbm.at[idx_v.at[j]], out_v.at[j], gsem)
        for j in range(NCHUNK)
    ]
    for cp in gathers:
        cp.wait()

    pltpu.sync_copy(out_v, out_hbm.at[pl.ds(wid * NCHUNK, NCHUNK)])


@jax.jit
def kernel(inputs, w_A, w_cond):
    inputs = inputs.astype(jnp.int32)
    w_A = w_A.astype(jnp.float32)
    w_cond = w_cond.astype(jnp.float32)

    table2, ab = pl.pallas_call(
        _lse_fold_body,
        in_specs=[
            pl.BlockSpec(memory_space=pl.ANY),
            pl.BlockSpec((N, 1), lambda: (0, 0)),
            pl.BlockSpec(memory_space=pl.ANY),
        ],
        out_specs=[
            pl.BlockSpec(memory_space=pl.ANY),
            pl.BlockSpec(memory_space=pl.ANY),
        ],
        out_shape=[
            jax.ShapeDtypeStruct((N * NPAD // 128, 128), jnp.float32),
            jax.ShapeDtypeStruct((B, 2), jnp.int32),
        ],
        input_output_aliases={2: 1},
        scratch_shapes=[
            pltpu.VMEM((N, N), jnp.float32),
            pltpu.VMEM((H0 * 8, 128), jnp.float32),
            pltpu.VMEM((H1 * 8, 128), jnp.float32),
            pltpu.SemaphoreType.DMA,
            pltpu.SemaphoreType.DMA,
        ],
    )(w_cond, w_A[:, None], inputs)

    in2 = ab.reshape(2 * B // CHUNK, CHUNK)
    t2_flat = table2.reshape(N * NPAD)

    sc_kernel = pl.kernel(
        _sc_body,
        out_type=jax.ShapeDtypeStruct((B // CHUNK, CHUNK), jnp.float32),
        mesh=plsc.VectorSubcoreMesh(core_axis_name="c", subcore_axis_name="s"),
        scratch_types=[
            pltpu.VMEM((IROWS, CHUNK), jnp.int32),
            pltpu.VMEM((NCHUNK, CHUNK), jnp.int32),
            pltpu.VMEM((NCHUNK, CHUNK), jnp.float32),
            pltpu.SemaphoreType.DMA,
            pltpu.SemaphoreType.DMA,
        ],
    )
    out2 = sc_kernel(in2, t2_flat)
    return out2.reshape(B)


# speedup vs baseline: 1.0224x; 1.0224x over previous
import jax
import jax.numpy as jnp
from jax import lax
from jax.experimental import pallas as pl
from jax.experimental.pallas import tpu as pltpu
from jax.experimental.pallas import tpu_sc as plsc

N = 1000
NPAD = 1024
B = 16384
NC = 2
NS = 16
NW = NC * NS
BPW = B // NW
LANES = 16
CHUNK = 128
NCHUNK = BPW // CHUNK
IROWS = 2 * BPW // CHUNK


H0 = 512
H1 = N - H0


def _lse_fold_body(wc_hbm, wa_ref, t2_hbm, wc_v, t2a_v, t2b_v,
                   sem_in, sem_out):
    cp_a = pltpu.async_copy(wc_hbm.at[pl.ds(0, H0)], wc_v.at[pl.ds(0, H0)],
                            sem_in)
    cp_b = pltpu.async_copy(wc_hbm.at[pl.ds(H0, H1)], wc_v.at[pl.ds(H0, H1)],
                            sem_in)

    wa_all = wa_ref[...]
    ma = jnp.max(wa_all)
    sa = jnp.sum(jnp.exp(wa_all - ma))
    lse_a = ma + jnp.log(sa)

    def fold_half(row0, nrows, stage_ref):
        x = wc_v[pl.ds(row0, nrows), :]
        m = jnp.max(x, axis=1, keepdims=True)
        s = jnp.sum(jnp.exp(x - m), axis=1, keepdims=True)
        lse_c = m + jnp.log(s)
        t2 = x + (wa_ref[pl.ds(row0, nrows), :] - lse_a - lse_c)
        t2p = jnp.concatenate(
            [t2, jnp.zeros((nrows, NPAD - N), jnp.float32)], axis=1)
        for rg in range(nrows // 8):
            for tj in range(NPAD // 128):
                stage_ref[pl.ds(rg * 64 + tj * 8, 8), :] = (
                    t2p[rg * 8:(rg + 1) * 8, tj * 128:(tj + 1) * 128])

    cp_a.wait()
    fold_half(0, H0, t2a_v)
    cp_oa = pltpu.async_copy(t2a_v, t2_hbm.at[pl.ds(0, H0 * 8)], sem_out)
    cp_b.wait()
    fold_half(H0, H1, t2b_v)
    cp_ob = pltpu.async_copy(t2b_v, t2_hbm.at[pl.ds(H0 * 8, H1 * 8)], sem_out)
    cp_oa.wait()
    cp_ob.wait()


def _lane_shuffle(v, idx):
    return lax.gather(
        v, idx[:, None],
        lax.GatherDimensionNumbers(
            offset_dims=(), collapsed_slice_dims=(0,), start_index_map=(0,)),
        (1,),
        mode=lax.GatherScatterMode.PROMISE_IN_BOUNDS)


def _sc_body(in_hbm, t2_hbm, out_hbm, iv_v, idx_v, out_v, sem, gsem):
    wid = lax.axis_index("s") * NC + lax.axis_index("c")

    pltpu.async_copy(in_hbm.at[pl.ds(wid * IROWS, IROWS)], iv_v, sem).wait()

    lane = lax.iota(jnp.int32, LANES)
    rot1 = lax.bitwise_and(lane + 1, LANES - 1)
    compact = lax.bitwise_and(lane * 2, LANES - 1)
    low_half = lane < (LANES // 2)

    for i in range(BPW // LANES):
        q1, t1 = (2 * i) // 8, (2 * i) % 8
        q2, t2 = (2 * i + 1) // 8, (2 * i + 1) % 8
        v1 = iv_v[q1, pl.ds(t1 * LANES, LANES)]
        v2 = iv_v[q2, pl.ds(t2 * LANES, LANES)]
        f1 = (v1 >> 3) * 8192 + (v1 & 7) * 128
        g1 = (v1 >> 7) * 1024 + (v1 & 127)
        f2 = (v2 >> 3) * 8192 + (v2 & 7) * 128
        g2 = (v2 >> 7) * 1024 + (v2 & 127)
        u1 = f1 + _lane_shuffle(g1, rot1)
        u2 = f2 + _lane_shuffle(g2, rot1)
        off = jnp.where(low_half,
                        _lane_shuffle(u1, compact),
                        _lane_shuffle(u2, compact))
        idx_v[i // 8, pl.ds((i % 8) * LANES, LANES)] = off

    gathers = [
        pltpu.async_copy(t2_hbm.at[idx_v.at[j]], out_v.at[j], gsem)
        for j in range(NCHUNK)
    ]
    for cp in gathers:
        cp.wait()

    pltpu.sync_copy(out_v, out_hbm.at[pl.ds(wid * NCHUNK, NCHUNK)])


@jax.jit
def kernel(inputs, w_A, w_cond):
    inputs = inputs.astype(jnp.int32)
    w_A = w_A.astype(jnp.float32)
    w_cond = w_cond.astype(jnp.float32)

    table2 = pl.pallas_call(
        _lse_fold_body,
        in_specs=[
            pl.BlockSpec(memory_space=pl.ANY),
            pl.BlockSpec((N, 1), lambda: (0, 0)),
        ],
        out_specs=pl.BlockSpec(memory_space=pl.ANY),
        out_shape=jax.ShapeDtypeStruct((N * NPAD // 128, 128), jnp.float32),
        scratch_shapes=[
            pltpu.VMEM((N, N), jnp.float32),
            pltpu.VMEM((H0 * 8, 128), jnp.float32),
            pltpu.VMEM((H1 * 8, 128), jnp.float32),
            pltpu.SemaphoreType.DMA,
            pltpu.SemaphoreType.DMA,
        ],
    )(w_cond, w_A[:, None])

    in2 = inputs.reshape(2 * B // CHUNK, CHUNK)
    t2_flat = table2.reshape(N * NPAD)

    sc_kernel = pl.kernel(
        _sc_body,
        out_type=jax.ShapeDtypeStruct((B // CHUNK, CHUNK), jnp.float32),
        mesh=plsc.VectorSubcoreMesh(core_axis_name="c", subcore_axis_name="s"),
        scratch_types=[
            pltpu.VMEM((IROWS, CHUNK), jnp.int32),
            pltpu.VMEM((NCHUNK, CHUNK), jnp.int32),
            pltpu.VMEM((NCHUNK, CHUNK), jnp.float32),
            pltpu.SemaphoreType.DMA,
            pltpu.SemaphoreType.DMA,
        ],
    )
    out2 = sc_kernel(in2, t2_flat)
    return out2.reshape(B)
